# Initial kernel scaffold; baseline (speedup 1.0000x reference)
import jax
import jax.numpy as jnp
from jax.experimental import pallas as pl


def kernel(x, edge_index, edge_attr, W1, b1, W2, b2):
    raise NotImplementedError("write your pallas kernel here")


# speedup vs baseline: 20.6840x; 20.6840x over previous
import functools

import jax
import jax.numpy as jnp
from jax import lax
from jax.experimental import pallas as pl
from jax.experimental.pallas import tpu as pltpu
from jax.experimental.pallas import tpu_sc as plsc

F32 = jnp.float32

NC = 2
NS = 16
L = 16
CH = 128


def _mesh():
    return plsc.VectorSubcoreMesh(core_axis_name="c", subcore_axis_name="s")


def _fill(buf, value, width):
    @pl.loop(0, width, step=L)
    def _(k):
        buf[pl.ds(k, L)] = jnp.full((L,), value, F32)


def _sc_deg(np_, c1):
    rows_per_tile = np_ // NS

    @functools.partial(
        pl.kernel,
        out_type=jax.ShapeDtypeStruct((NC, np_), F32),
        mesh=_mesh(),
        scratch_types=[
            pltpu.VMEM((c1, CH), jnp.int32),
            pltpu.VMEM((CH,), F32),
            pltpu.VMEM_SHARED((np_,), F32),
        ],
    )
    def k(dst_hbm, out_hbm, didx, buf, acc):
        c = lax.axis_index("c")
        s = lax.axis_index("s")
        pltpu.sync_copy(dst_hbm.at[c, s], didx)
        _fill(buf, 0.0, CH)
        for i in range(rows_per_tile // CH):
            pltpu.sync_copy(buf, acc.at[pl.ds(s * rows_per_tile + i * CH, CH)])
        _fill(buf, 1.0, CH)
        plsc.subcore_barrier()

        @pl.loop(0, c1)
        def _(j):
            pltpu.sync_copy(buf, acc.at[didx.at[j]], add=True)

        plsc.subcore_barrier()
        sl = pl.ds(s * rows_per_tile, rows_per_tile)
        pltpu.sync_copy(acc.at[sl], out_hbm.at[c, sl])

    return k


def _sc_agg(n, d, np_, c1):
    rows_per_tile = np_ // NS

    @functools.partial(
        pl.kernel,
        out_type=jax.ShapeDtypeStruct((NC, np_, d), F32),
        mesh=_mesh(),
        scratch_types=[
            pltpu.VMEM((c1, CH), jnp.int32),
            pltpu.VMEM((c1, CH), jnp.int32),
            pltpu.VMEM((CH, d), F32),
            pltpu.VMEM_SHARED((np_, d), F32),
            pltpu.SemaphoreType.DMA,
        ],
    )
    def k(gx_hbm, src_hbm, dst_hbm, out_hbm, sidx, didx, rows, acc, sem):
        c = lax.axis_index("c")
        s = lax.axis_index("s")
        pltpu.sync_copy(src_hbm.at[c, s], sidx)
        pltpu.sync_copy(dst_hbm.at[c, s], didx)

        @pl.loop(0, CH)
        def _(r):
            for kk in range(d // L):
                rows[r, pl.ds(kk * L, L)] = jnp.zeros((L,), F32)

        for i in range(rows_per_tile // CH):
            pltpu.sync_copy(rows, acc.at[pl.ds(s * rows_per_tile + i * CH, CH)])
        plsc.subcore_barrier()

        @pl.loop(0, c1)
        def _(j):
            pltpu.async_copy(gx_hbm.at[sidx.at[j]], rows, sem).wait()
            pltpu.sync_copy(rows, acc.at[didx.at[j]], add=True)

        plsc.subcore_barrier()
        sl = pl.ds(s * rows_per_tile, rows_per_tile)
        pltpu.sync_copy(acc.at[sl], out_hbm.at[c, sl])

    return k


def _sc_agg1(n, np_, c1):
    rows_per_tile = np_ // NS

    @functools.partial(
        pl.kernel,
        out_type=jax.ShapeDtypeStruct((NC, np_), F32),
        mesh=_mesh(),
        scratch_types=[
            pltpu.VMEM((c1, CH), jnp.int32),
            pltpu.VMEM((c1, CH), jnp.int32),
            pltpu.VMEM((CH,), F32),
            pltpu.VMEM_SHARED((np_,), F32),
            pltpu.SemaphoreType.DMA,
        ],
    )
    def k(g2_hbm, src_hbm, dst_hbm, out_hbm, sidx, didx, vals, acc, sem):
        c = lax.axis_index("c")
        s = lax.axis_index("s")
        pltpu.sync_copy(src_hbm.at[c, s], sidx)
        pltpu.sync_copy(dst_hbm.at[c, s], didx)
        _fill(vals, 0.0, CH)
        for i in range(rows_per_tile // CH):
            pltpu.sync_copy(vals, acc.at[pl.ds(s * rows_per_tile + i * CH, CH)])
        plsc.subcore_barrier()

        @pl.loop(0, c1)
        def _(j):
            pltpu.async_copy(g2_hbm.at[sidx.at[j]], vals, sem).wait()
            pltpu.sync_copy(vals, acc.at[didx.at[j]], add=True)

        plsc.subcore_barrier()
        sl = pl.ds(s * rows_per_tile, rows_per_tile)
        pltpu.sync_copy(acc.at[sl], out_hbm.at[c, sl])

    return k


def _tc_prep(dp, x, n):
    def body(dp_ref, x_ref, dis_ref, gx_ref):
        deg = dp_ref[0] + dp_ref[1] + 1.0
        dis = lax.rsqrt(deg)
        dis_ref[...] = dis
        gx_ref[...] = dis[:n] * x_ref[...]

    np_ = dp.shape[1]
    return pl.pallas_call(
        body,
        out_shape=(
            jax.ShapeDtypeStruct((np_, 1), F32),
            jax.ShapeDtypeStruct(x.shape, F32),
        ),
    )(dp, x)


def _tc_mid(sp, dis, gx, w1, b1, w2, n):
    def body(sp_ref, dis_ref, gx_ref, w1_ref, b1_ref, w2_ref, g2_ref):
        s_sum = sp_ref[0, :n] + sp_ref[1, :n]
        dis_n = dis_ref[:n]
        t = dis_n * s_sum + (dis_n * dis_n) * gx_ref[...]
        x1 = jax.nn.sigmoid(
            jnp.dot(t, w1_ref[...], preferred_element_type=F32,
                    precision=lax.Precision.HIGHEST) + b1_ref[...]
        )
        h2 = jnp.dot(x1, w2_ref[...], preferred_element_type=F32,
                     precision=lax.Precision.HIGHEST)
        g2_ref[...] = dis_n * h2

    return pl.pallas_call(
        body,
        out_shape=jax.ShapeDtypeStruct((n, 1), F32),
    )(sp, dis, gx, w1, b1, w2)


def _tc_final(zp, dis, g2, b2, n):
    def body(zp_ref, dis_ref, g2_ref, b2_ref, out_ref):
        z = zp_ref[0, :n] + zp_ref[1, :n]
        dis_n = dis_ref[:n]
        out_ref[...] = jax.nn.sigmoid(
            dis_n * z + (dis_n * dis_n) * g2_ref[...] + b2_ref[...]
        )

    return pl.pallas_call(
        body,
        out_shape=jax.ShapeDtypeStruct((n, 1), F32),
    )(zp, dis, g2, b2)


def kernel(x, edge_index, edge_attr, W1, b1, W2, b2):
    n, d = x.shape
    e = edge_index.shape[1]

    per_tile = -(-e // (NC * NS * CH)) * CH
    e_pad = per_tile * NC * NS
    c1 = per_tile // CH
    np_ = -(-(n + 1) // (NS * CH)) * NS * CH

    src = edge_index[0]
    dst = edge_index[1]
    if e_pad > e:
        src = jnp.concatenate([src, jnp.zeros((e_pad - e,), jnp.int32)])
        dst = jnp.concatenate([dst, jnp.full((e_pad - e,), n, jnp.int32)])
    src_r = src.reshape(NC, NS, c1, CH)
    dst_r = dst.reshape(NC, NS, c1, CH)

    deg_parts = _sc_deg(np_, c1)(dst_r)
    dis, gx = _tc_prep(deg_parts.reshape(NC, np_, 1), x, n)
    s_parts = _sc_agg(n, d, np_, c1)(gx, src_r, dst_r)
    g2 = _tc_mid(s_parts, dis, gx, W1, b1.reshape(1, d), W2, n)
    z_parts = _sc_agg1(n, np_, c1)(g2.reshape(n), src_r, dst_r)
    return _tc_final(z_parts.reshape(NC, np_, 1), dis, g2, b2.reshape(1, 1), n)


# speedup vs baseline: 22.7379x; 1.0993x over previous
import dataclasses
import functools

import jax
import jax.numpy as jnp
from jax import lax
from jax.experimental import pallas as pl
from jax.experimental.pallas import tpu as pltpu
from jax.experimental.pallas import tpu_sc as plsc

F32 = jnp.float32

NC = 2
NS = 16
L = 16
CH = 128


def _mesh():
    return plsc.VectorSubcoreMesh(core_axis_name="c", subcore_axis_name="s")


def _no_layout_params():
    cp = pltpu.CompilerParams()
    if "needs_layout_passes" in pltpu.CompilerParams.__dataclass_fields__:
        cp = dataclasses.replace(cp, needs_layout_passes=False)
    return cp


def _flat_tiling_params():
    return dataclasses.replace(pltpu.CompilerParams(),
                               use_tc_tiling_on_sc=False)


def _fill(buf, value, width):
    @pl.loop(0, width, step=L)
    def _(k):
        buf[pl.ds(k, L)] = jnp.full((L,), value, F32)


def _reduce_tiles(z_v, z_sh, out_hbm, c, s, np_, tmp, zacc):
    rows_per_tile = np_ // NS
    pltpu.sync_copy(z_v, z_sh.at[s])
    plsc.subcore_barrier()
    _fill(zacc, 0.0, rows_per_tile)

    @pl.loop(0, NS)
    def _(ss):
        pltpu.sync_copy(z_sh.at[ss, pl.ds(s * rows_per_tile, rows_per_tile)],
                        tmp)

        @pl.loop(0, rows_per_tile, step=L)
        def _(k):
            zacc[pl.ds(k, L)] += tmp[pl.ds(k, L)]

    pltpu.sync_copy(zacc, out_hbm.at[c, pl.ds(s * rows_per_tile,
                                              rows_per_tile)])


def _sc_deg(np_, c1):
    rows_per_tile = np_ // NS

    @functools.partial(
        pl.kernel,
        out_type=jax.ShapeDtypeStruct((NC, np_), F32),
        mesh=_mesh(),
        scratch_types=[
            pltpu.VMEM((c1, CH), jnp.int32),
            pltpu.VMEM((np_,), F32),
            pltpu.VMEM_SHARED((NS, np_), F32),
            pltpu.VMEM((np_ // NS,), F32),
            pltpu.VMEM((np_ // NS,), F32),
        ],
        compiler_params=_no_layout_params(),
    )
    def k(dst_hbm, out_hbm, didx, deg_v, z_sh, tmp, zacc):
        c = lax.axis_index("c")
        s = lax.axis_index("s")
        pltpu.sync_copy(dst_hbm.at[c, s], didx)
        _fill(deg_v, 0.0, np_)
        ones = jnp.ones((L,), F32)

        @pl.loop(0, c1)
        def _(j):
            for kk in range(CH // L):
                d16 = didx[j, pl.ds(kk * L, L)]
                plsc.addupdate_scatter(deg_v, [d16], ones)

        _reduce_tiles(deg_v, z_sh, out_hbm, c, s, np_, tmp, zacc)

    return k


def _sc_agg(n, d, np_, c2):
    dh = d // NC
    rows_per_tile = np_ // NS

    @functools.partial(
        pl.kernel,
        out_type=jax.ShapeDtypeStruct((NC, np_, dh), F32),
        mesh=_mesh(),
        scratch_types=[
            pltpu.VMEM((c2, CH), jnp.int32),
            pltpu.VMEM((c2, CH), jnp.int32),
            pltpu.VMEM((CH, dh), F32),
            pltpu.VMEM((CH, dh), F32),
            pltpu.VMEM_SHARED((np_, dh), F32),
            pltpu.SemaphoreType.DMA,
            pltpu.SemaphoreType.DMA,
            pltpu.SemaphoreType.DMA,
            pltpu.SemaphoreType.DMA,
        ],
        compiler_params=_flat_tiling_params(),
    )
    def k(gx_hbm, src_hbm, dst_hbm, out_hbm, sidx, didx, rows0, rows1, acc,
          gsem0, gsem1, ssem0, ssem1):
        c = lax.axis_index("c")
        s = lax.axis_index("s")
        rows = (rows0, rows1)
        gsem = (gsem0, gsem1)
        ssem = (ssem0, ssem1)
        gxc = gx_hbm.at[c]
        pltpu.sync_copy(src_hbm.at[s], sidx)
        pltpu.sync_copy(dst_hbm.at[s], didx)

        @pl.loop(0, CH)
        def _(r):
            for kk in range(dh // L):
                rows0[r, pl.ds(kk * L, L)] = jnp.zeros((L,), F32)

        for i in range(rows_per_tile // CH):
            pltpu.sync_copy(rows0,
                            acc.at[pl.ds(s * rows_per_tile + i * CH, CH)])
        plsc.subcore_barrier()

        for t in range(2):
            pltpu.async_copy(gxc.at[sidx.at[t]], rows[t], gsem[t])

        @pl.loop(0, c2, step=2)
        def _(j):
            for t in range(2):
                jj = j + t
                pltpu.make_async_copy(gxc.at[sidx.at[jj]], rows[t],
                                      gsem[t]).wait()
                pltpu.async_copy(rows[t], acc.at[didx.at[jj]], ssem[t],
                                 add=True).wait()

                @pl.when(jj + 2 < c2)
                def _():
                    pltpu.async_copy(gxc.at[sidx.at[jj + 2]], rows[t],
                                     gsem[t])

        plsc.subcore_barrier()
        sl = pl.ds(s * rows_per_tile, rows_per_tile)
        pltpu.sync_copy(acc.at[sl], out_hbm.at[c, sl])

    return k


def _sc_agg1(n, np_, c1):
    @functools.partial(
        pl.kernel,
        out_type=jax.ShapeDtypeStruct((NC, np_), F32),
        mesh=_mesh(),
        scratch_types=[
            pltpu.VMEM((c1, CH), jnp.int32),
            pltpu.VMEM((c1, CH), jnp.int32),
            pltpu.VMEM((n,), F32),
            pltpu.VMEM((np_,), F32),
            pltpu.VMEM_SHARED((NS, np_), F32),
            pltpu.VMEM((np_ // NS,), F32),
            pltpu.VMEM((np_ // NS,), F32),
        ],
        compiler_params=_no_layout_params(),
    )
    def k(g2_hbm, src_hbm, dst_hbm, out_hbm, sidx, didx, g2_v, z_v, z_sh,
          tmp, zacc):
        c = lax.axis_index("c")
        s = lax.axis_index("s")
        pltpu.sync_copy(src_hbm.at[c, s], sidx)
        pltpu.sync_copy(dst_hbm.at[c, s], didx)
        pltpu.sync_copy(g2_hbm, g2_v)
        _fill(z_v, 0.0, np_)

        @pl.loop(0, c1)
        def _(j):
            for kk in range(CH // L):
                s16 = sidx[j, pl.ds(kk * L, L)]
                d16 = didx[j, pl.ds(kk * L, L)]
                v = plsc.load_gather(g2_v, [s16])
                plsc.addupdate_scatter(z_v, [d16], v)

        _reduce_tiles(z_v, z_sh, out_hbm, c, s, np_, tmp, zacc)

    return k


def _tc_prep(dp, x, n):
    d = x.shape[1]
    dh = d // NC

    def body(dp_ref, x_ref, dis_ref, gx_ref):
        deg = dp_ref[0] + dp_ref[1] + 1.0
        dis = lax.rsqrt(deg)
        dis_ref[...] = dis
        gxs = dis[:n] * x_ref[...]
        gx_ref[0] = gxs[:, :dh]
        gx_ref[1] = gxs[:, dh:]

    np_ = dp.shape[1]
    return pl.pallas_call(
        body,
        out_shape=(
            jax.ShapeDtypeStruct((np_, 1), F32),
            jax.ShapeDtypeStruct((NC, n, dh), F32),
        ),
    )(dp, x)


def _tc_mid(sp, dis, gx, w1, b1, w2, n):
    def body(sp_ref, dis_ref, gx_ref, w1_ref, b1_ref, w2_ref, g2_ref):
        s_sum = jnp.concatenate([sp_ref[0, :n], sp_ref[1, :n]], axis=1)
        gx_full = jnp.concatenate([gx_ref[0], gx_ref[1]], axis=1)
        dis_n = dis_ref[:n]
        t = dis_n * s_sum + (dis_n * dis_n) * gx_full
        x1 = jax.nn.sigmoid(
            jnp.dot(t, w1_ref[...], preferred_element_type=F32,
                    precision=lax.Precision.HIGHEST) + b1_ref[...]
        )
        h2 = jnp.dot(x1, w2_ref[...], preferred_element_type=F32,
                     precision=lax.Precision.HIGHEST)
        g2_ref[...] = dis_n * h2

    return pl.pallas_call(
        body,
        out_shape=jax.ShapeDtypeStruct((n, 1), F32),
    )(sp, dis, gx, w1, b1, w2)


def _tc_final(zp, dis, g2, b2, n):
    def body(zp_ref, dis_ref, g2_ref, b2_ref, out_ref):
        z = zp_ref[0, :n] + zp_ref[1, :n]
        dis_n = dis_ref[:n]
        out_ref[...] = jax.nn.sigmoid(
            dis_n * z + (dis_n * dis_n) * g2_ref[...] + b2_ref[...]
        )

    return pl.pallas_call(
        body,
        out_shape=jax.ShapeDtypeStruct((n, 1), F32),
    )(zp, dis, g2, b2)


def kernel(x, edge_index, edge_attr, W1, b1, W2, b2):
    n, d = x.shape
    e = edge_index.shape[1]

    c1 = -(-e // (NC * NS * CH))
    if c1 % 2:
        c1 += 1
    e_pad = c1 * NC * NS * CH
    c2 = 2 * c1
    np_ = -(-(n + 1) // (NS * CH)) * NS * CH

    src = edge_index[0]
    dst = edge_index[1]
    if e_pad > e:
        src = jnp.concatenate([src, jnp.zeros((e_pad - e,), jnp.int32)])
        dst = jnp.concatenate([dst, jnp.full((e_pad - e,), n, jnp.int32)])
    src_r = src.reshape(NC, NS, c1, CH)
    dst_r = dst.reshape(NC, NS, c1, CH)
    src_f = src.reshape(NS, c2, CH)
    dst_f = dst.reshape(NS, c2, CH)

    deg_parts = _sc_deg(np_, c1)(dst_r)
    dis, gx = _tc_prep(deg_parts.reshape(NC, np_, 1), x, n)
    s_parts = _sc_agg(n, d, np_, c2)(gx, src_f, dst_f)
    g2 = _tc_mid(s_parts, dis, gx, W1, b1.reshape(1, d), W2, n)
    z_parts = _sc_agg1(n, np_, c1)(g2.reshape(n), src_r, dst_r)
    return _tc_final(z_parts.reshape(NC, np_, 1), dis, g2, b2.reshape(1, 1), n)


# speedup vs baseline: 31.0559x; 1.3658x over previous
import dataclasses
import functools

import jax
import jax.numpy as jnp
from jax import lax
from jax.experimental import pallas as pl
from jax.experimental.pallas import tpu as pltpu
from jax.experimental.pallas import tpu_sc as plsc

F32 = jnp.float32

NC = 2
NS = 16
L = 16
CH = 128


def _mesh():
    return plsc.VectorSubcoreMesh(core_axis_name="c", subcore_axis_name="s")


def _no_layout_params():
    cp = pltpu.CompilerParams()
    if "needs_layout_passes" in pltpu.CompilerParams.__dataclass_fields__:
        cp = dataclasses.replace(cp, needs_layout_passes=False)
    return cp


def _flat_tiling_params():
    return dataclasses.replace(pltpu.CompilerParams(),
                               use_tc_tiling_on_sc=False,
                               internal_scratch_in_bytes=1024 * 1024)


def _fill(buf, value, width):
    @pl.loop(0, width, step=L)
    def _(k):
        buf[pl.ds(k, L)] = jnp.full((L,), value, F32)


def _reduce_tiles(z_v, z_sh, out_hbm, c, s, np_, tmp, zacc):
    rows_per_tile = np_ // NS
    pltpu.sync_copy(z_v, z_sh.at[s])
    plsc.subcore_barrier()
    _fill(zacc, 0.0, rows_per_tile)

    @pl.loop(0, NS)
    def _(ss):
        pltpu.sync_copy(z_sh.at[ss, pl.ds(s * rows_per_tile, rows_per_tile)],
                        tmp)

        @pl.loop(0, rows_per_tile, step=L)
        def _(k):
            zacc[pl.ds(k, L)] += tmp[pl.ds(k, L)]

    pltpu.sync_copy(zacc, out_hbm.at[c, pl.ds(s * rows_per_tile,
                                              rows_per_tile)])


def _sc_deg(np_, c1):
    rows_per_tile = np_ // NS

    @functools.partial(
        pl.kernel,
        out_type=jax.ShapeDtypeStruct((NC, np_), F32),
        mesh=_mesh(),
        scratch_types=[
            pltpu.VMEM((c1, CH), jnp.int32),
            pltpu.VMEM((np_,), F32),
            pltpu.VMEM_SHARED((NS, np_), F32),
            pltpu.VMEM((np_ // NS,), F32),
            pltpu.VMEM((np_ // NS,), F32),
        ],
        compiler_params=_no_layout_params(),
    )
    def k(dst_hbm, out_hbm, didx, deg_v, z_sh, tmp, zacc):
        c = lax.axis_index("c")
        s = lax.axis_index("s")
        pltpu.sync_copy(dst_hbm.at[c, s], didx)
        _fill(deg_v, 0.0, np_)
        ones = jnp.ones((L,), F32)

        @pl.loop(0, c1)
        def _(j):
            for kk in range(CH // L):
                d16 = didx[j, pl.ds(kk * L, L)]
                plsc.addupdate_scatter(deg_v, [d16], ones)

        _reduce_tiles(deg_v, z_sh, out_hbm, c, s, np_, tmp, zacc)

    return k


def _sc_agg(n, d, np_, c2):
    nq = 2 * NC
    dq = d // nq
    rows_per_tile = np_ // NS
    nrt = n // NS

    @functools.partial(
        pl.kernel,
        out_type=jax.ShapeDtypeStruct((nq, np_, dq), F32),
        mesh=_mesh(),
        scratch_types=[
            pltpu.VMEM((c2, CH), jnp.int32),
            pltpu.VMEM((c2, CH), jnp.int32),
            pltpu.VMEM((CH, dq), F32),
            pltpu.VMEM((CH, dq), F32),
            pltpu.VMEM_SHARED((n, dq), F32),
            pltpu.VMEM_SHARED((np_, dq), F32),
            pltpu.SemaphoreType.DMA,
            pltpu.SemaphoreType.DMA,
            pltpu.SemaphoreType.DMA,
            pltpu.SemaphoreType.DMA,
        ],
        compiler_params=_flat_tiling_params(),
    )
    def k(gx_hbm, src_hbm, dst_hbm, out_hbm, sidx, didx, rows0, rows1, gx_sp,
          acc, gsem0, gsem1, ssem0, ssem1):
        c = lax.axis_index("c")
        s = lax.axis_index("s")
        rows = (rows0, rows1)
        gsem = (gsem0, gsem1)
        ssem = (ssem0, ssem1)
        pltpu.sync_copy(src_hbm.at[s], sidx)
        pltpu.sync_copy(dst_hbm.at[s], didx)
        tsl = pl.ds(s * nrt, nrt)
        sl = pl.ds(s * rows_per_tile, rows_per_tile)

        for p in range(nq // NC):
            q = c * (nq // NC) + p
            pltpu.sync_copy(gx_hbm.at[q, tsl], gx_sp.at[tsl])

            @pl.loop(0, CH)
            def _(r):
                for kk in range(dq // L):
                    rows0[r, pl.ds(kk * L, L)] = jnp.zeros((L,), F32)

            for i in range(rows_per_tile // CH):
                pltpu.sync_copy(rows0,
                                acc.at[pl.ds(s * rows_per_tile + i * CH, CH)])
            plsc.subcore_barrier()

            for t in range(2):
                pltpu.async_copy(gx_sp.at[sidx.at[t]], rows[t], gsem[t])

            @pl.loop(0, c2, step=2)
            def _(j):
                for t in range(2):
                    jj = j + t
                    pltpu.make_async_copy(gx_sp.at[sidx.at[jj]], rows[t],
                                          gsem[t]).wait()
                    pltpu.async_copy(rows[t], acc.at[didx.at[jj]], ssem[t],
                                     add=True).wait()

                    @pl.when(jj + 2 < c2)
                    def _():
                        pltpu.async_copy(gx_sp.at[sidx.at[jj + 2]], rows[t],
                                         gsem[t])

            plsc.subcore_barrier()
            pltpu.sync_copy(acc.at[sl], out_hbm.at[q, sl])
            if p + 1 < nq // NC:
                plsc.subcore_barrier()

    return k


def _sc_agg1(n, np_, c1):
    @functools.partial(
        pl.kernel,
        out_type=jax.ShapeDtypeStruct((NC, np_), F32),
        mesh=_mesh(),
        scratch_types=[
            pltpu.VMEM((c1, CH), jnp.int32),
            pltpu.VMEM((c1, CH), jnp.int32),
            pltpu.VMEM((n,), F32),
            pltpu.VMEM((np_,), F32),
            pltpu.VMEM_SHARED((NS, np_), F32),
            pltpu.VMEM((np_ // NS,), F32),
            pltpu.VMEM((np_ // NS,), F32),
        ],
        compiler_params=_no_layout_params(),
    )
    def k(g2_hbm, src_hbm, dst_hbm, out_hbm, sidx, didx, g2_v, z_v, z_sh,
          tmp, zacc):
        c = lax.axis_index("c")
        s = lax.axis_index("s")
        pltpu.sync_copy(src_hbm.at[c, s], sidx)
        pltpu.sync_copy(dst_hbm.at[c, s], didx)
        pltpu.sync_copy(g2_hbm, g2_v)
        _fill(z_v, 0.0, np_)

        @pl.loop(0, c1)
        def _(j):
            for kk in range(CH // L):
                s16 = sidx[j, pl.ds(kk * L, L)]
                d16 = didx[j, pl.ds(kk * L, L)]
                v = plsc.load_gather(g2_v, [s16])
                plsc.addupdate_scatter(z_v, [d16], v)

        _reduce_tiles(z_v, z_sh, out_hbm, c, s, np_, tmp, zacc)

    return k


def _tc_prep(dp, x, n):
    d = x.shape[1]
    nq = 2 * NC
    dq = d // nq

    def body(dp_ref, x_ref, dis_ref, gx_ref):
        deg = dp_ref[0] + dp_ref[1] + 1.0
        dis = lax.rsqrt(deg)
        dis_ref[...] = dis
        gxs = dis[:n] * x_ref[...]
        for q in range(nq):
            gx_ref[q] = gxs[:, q * dq:(q + 1) * dq]

    np_ = dp.shape[1]
    return pl.pallas_call(
        body,
        out_shape=(
            jax.ShapeDtypeStruct((np_, 1), F32),
            jax.ShapeDtypeStruct((nq, n, dq), F32),
        ),
    )(dp, x)


def _tc_mid(sp, dis, gx, w1, b1, w2, n):
    nq, np_, dq = sp.shape
    d = nq * dq
    bn = 2000
    assert n % bn == 0

    def body(sp_ref, dis_ref, gx_ref, w1_ref, b1_ref, w2_ref, g2_ref):
        s_sum = jnp.concatenate([sp_ref[q] for q in range(nq)], axis=1)
        gx_full = jnp.concatenate([gx_ref[q] for q in range(nq)], axis=1)
        dis_n = dis_ref[...]
        t = dis_n * s_sum + (dis_n * dis_n) * gx_full
        x1 = jax.nn.sigmoid(
            jnp.dot(t, w1_ref[...], preferred_element_type=F32,
                    precision=lax.Precision.HIGHEST) + b1_ref[...]
        )
        h2 = jnp.dot(x1, w2_ref[...], preferred_element_type=F32,
                     precision=lax.Precision.HIGHEST)
        g2_ref[...] = dis_n * h2

    return pl.pallas_call(
        body,
        grid=(n // bn,),
        in_specs=[
            pl.BlockSpec((nq, bn, dq), lambda i: (0, i, 0)),
            pl.BlockSpec((bn, 1), lambda i: (i, 0)),
            pl.BlockSpec((nq, bn, dq), lambda i: (0, i, 0)),
            pl.BlockSpec((d, d), lambda i: (0, 0)),
            pl.BlockSpec((1, d), lambda i: (0, 0)),
            pl.BlockSpec((d, 1), lambda i: (0, 0)),
        ],
        out_specs=pl.BlockSpec((bn, 1), lambda i: (i, 0)),
        out_shape=jax.ShapeDtypeStruct((n, 1), F32),
    )(sp, dis, gx, w1, b1, w2)


def _tc_final(zp, dis, g2, b2, n):
    def body(zp_ref, dis_ref, g2_ref, b2_ref, out_ref):
        z = zp_ref[0, :n] + zp_ref[1, :n]
        dis_n = dis_ref[:n]
        out_ref[...] = jax.nn.sigmoid(
            dis_n * z + (dis_n * dis_n) * g2_ref[...] + b2_ref[...]
        )

    return pl.pallas_call(
        body,
        out_shape=jax.ShapeDtypeStruct((n, 1), F32),
    )(zp, dis, g2, b2)


def kernel(x, edge_index, edge_attr, W1, b1, W2, b2):
    n, d = x.shape
    e = edge_index.shape[1]

    c1 = -(-e // (NC * NS * CH))
    if c1 % 2:
        c1 += 1
    e_pad = c1 * NC * NS * CH
    c2 = 2 * c1
    np_ = -(-(n + 1) // (NS * CH)) * NS * CH

    src = edge_index[0]
    dst = edge_index[1]
    if e_pad > e:
        src = jnp.concatenate([src, jnp.zeros((e_pad - e,), jnp.int32)])
        dst = jnp.concatenate([dst, jnp.full((e_pad - e,), n, jnp.int32)])
    src_r = src.reshape(NC, NS, c1, CH)
    dst_r = dst.reshape(NC, NS, c1, CH)
    src_f = src.reshape(NS, c2, CH)
    dst_f = dst.reshape(NS, c2, CH)

    deg_parts = _sc_deg(np_, c1)(dst_r)
    dis, gx = _tc_prep(deg_parts.reshape(NC, np_, 1), x, n)
    s_parts = _sc_agg(n, d, np_, c2)(gx, src_f, dst_f)
    g2 = _tc_mid(s_parts, dis, gx, W1, b1.reshape(1, d), W2, n)
    z_parts = _sc_agg1(n, np_, c1)(g2.reshape(n), src_r, dst_r)
    return _tc_final(z_parts.reshape(NC, np_, 1), dis, g2, b2.reshape(1, 1), n)


# speedup vs baseline: 31.5458x; 1.0158x over previous
import dataclasses
import functools

import jax
import jax.numpy as jnp
from jax import lax
from jax.experimental import pallas as pl
from jax.experimental.pallas import tpu as pltpu
from jax.experimental.pallas import tpu_sc as plsc

F32 = jnp.float32

NC = 2
NS = 16
L = 16
CH = 128


def _mesh():
    return plsc.VectorSubcoreMesh(core_axis_name="c", subcore_axis_name="s")


def _no_layout_params():
    cp = pltpu.CompilerParams()
    if "needs_layout_passes" in pltpu.CompilerParams.__dataclass_fields__:
        cp = dataclasses.replace(cp, needs_layout_passes=False)
    return dataclasses.replace(cp, use_tc_tiling_on_sc=False)


def _flat_tiling_params():
    return dataclasses.replace(pltpu.CompilerParams(),
                               use_tc_tiling_on_sc=False,
                               internal_scratch_in_bytes=1024 * 1024)


def _fill(buf, value, width):
    @pl.loop(0, width, step=L)
    def _(k):
        buf[pl.ds(k, L)] = jnp.full((L,), value, F32)


def _reduce_tiles(z_v, z_sh, out_hbm, c, s, np_, tmp, zacc):
    rows_per_tile = np_ // NS
    pltpu.sync_copy(z_v, z_sh.at[s])
    plsc.subcore_barrier()
    _fill(zacc, 0.0, rows_per_tile)

    @pl.loop(0, NS)
    def _(ss):
        pltpu.sync_copy(z_sh.at[ss, pl.ds(s * rows_per_tile, rows_per_tile)],
                        tmp)

        @pl.loop(0, rows_per_tile, step=L)
        def _(k):
            zacc[pl.ds(k, L)] += tmp[pl.ds(k, L)]

    pltpu.sync_copy(zacc, out_hbm.at[c, pl.ds(s * rows_per_tile,
                                              rows_per_tile)])


def _sc_deg(np_, c1):
    rows_per_tile = np_ // NS

    @functools.partial(
        pl.kernel,
        out_type=jax.ShapeDtypeStruct((NC, np_), F32),
        mesh=_mesh(),
        scratch_types=[
            pltpu.VMEM((c1, CH), jnp.int32),
            pltpu.VMEM((np_,), F32),
            pltpu.VMEM_SHARED((NS, np_), F32),
            pltpu.VMEM((np_ // NS,), F32),
            pltpu.VMEM((np_ // NS,), F32),
        ],
        compiler_params=_no_layout_params(),
    )
    def k(dst_hbm, out_hbm, didx, deg_v, z_sh, tmp, zacc):
        c = lax.axis_index("c")
        s = lax.axis_index("s")
        pltpu.sync_copy(dst_hbm.at[c, s], didx)
        _fill(deg_v, 0.0, np_)
        ones = jnp.ones((L,), F32)

        @pl.loop(0, c1)
        def _(j):
            for kk in range(CH // L):
                d16 = didx[j, pl.ds(kk * L, L)]
                plsc.addupdate_scatter(deg_v, [d16], ones)

        _reduce_tiles(deg_v, z_sh, out_hbm, c, s, np_, tmp, zacc)

    return k


def _sc_agg(n, d, np_, c2):
    nq = 2 * NC
    dq = d // nq
    rows_per_tile = np_ // NS
    nrt = n // NS

    @functools.partial(
        pl.kernel,
        out_type=jax.ShapeDtypeStruct((nq, np_, dq), F32),
        mesh=_mesh(),
        scratch_types=[
            pltpu.VMEM((c2, CH), jnp.int32),
            pltpu.VMEM((c2, CH), jnp.int32),
            pltpu.VMEM((CH, dq), F32),
            pltpu.VMEM((CH, dq), F32),
            pltpu.VMEM_SHARED((n, dq), F32),
            pltpu.VMEM_SHARED((np_, dq), F32),
            pltpu.SemaphoreType.DMA,
            pltpu.SemaphoreType.DMA,
            pltpu.SemaphoreType.DMA,
            pltpu.SemaphoreType.DMA,
        ],
        compiler_params=_flat_tiling_params(),
    )
    def k(gx_hbm, src_hbm, dst_hbm, out_hbm, sidx, didx, rows0, rows1, gx_sp,
          acc, gsem0, gsem1, ssem0, ssem1):
        c = lax.axis_index("c")
        s = lax.axis_index("s")
        rows = (rows0, rows1)
        gsem = (gsem0, gsem1)
        ssem = (ssem0, ssem1)
        pltpu.sync_copy(src_hbm.at[s], sidx)
        pltpu.sync_copy(dst_hbm.at[s], didx)
        tsl = pl.ds(s * nrt, nrt)
        sl = pl.ds(s * rows_per_tile, rows_per_tile)

        for p in range(nq // NC):
            q = c * (nq // NC) + p
            pltpu.sync_copy(gx_hbm.at[q, tsl], gx_sp.at[tsl])

            @pl.loop(0, CH)
            def _(r):
                for kk in range(dq // L):
                    rows0[r, pl.ds(kk * L, L)] = jnp.zeros((L,), F32)

            for i in range(rows_per_tile // CH):
                pltpu.sync_copy(rows0,
                                acc.at[pl.ds(s * rows_per_tile + i * CH, CH)])
            plsc.subcore_barrier()

            for t in range(2):
                pltpu.async_copy(gx_sp.at[sidx.at[t]], rows[t], gsem[t])

            @pl.loop(0, c2, step=2)
            def _(j):
                for t in range(2):
                    jj = j + t
                    pltpu.make_async_copy(gx_sp.at[sidx.at[jj]], rows[t],
                                          gsem[t]).wait()
                    pltpu.async_copy(rows[t], acc.at[didx.at[jj]], ssem[t],
                                     add=True).wait()

                    @pl.when(jj + 2 < c2)
                    def _():
                        pltpu.async_copy(gx_sp.at[sidx.at[jj + 2]], rows[t],
                                         gsem[t])

            plsc.subcore_barrier()
            pltpu.sync_copy(acc.at[sl], out_hbm.at[q, sl])
            if p + 1 < nq // NC:
                plsc.subcore_barrier()

    return k


def _sc_agg1(n, np_, c2):
    rpt2 = np_ // (NC * NS)

    @functools.partial(
        pl.kernel,
        out_type=jax.ShapeDtypeStruct((np_,), F32),
        mesh=_mesh(),
        scratch_types=[
            pltpu.VMEM((c2, CH), jnp.int32),
            pltpu.VMEM((c2, CH), jnp.int32),
            pltpu.VMEM((np_,), F32),
            pltpu.VMEM((np_,), F32),
            pltpu.VMEM_SHARED((NS, np_), F32),
            pltpu.VMEM((rpt2,), F32),
            pltpu.VMEM((rpt2,), F32),
            pltpu.VMEM((L,), F32),
        ],
        compiler_params=_no_layout_params(),
    )
    def k(g2_hbm, dis_hbm, b2_hbm, src_hbm, dst_hbm, out_hbm, sidx, didx,
          g2_v, z_v, z_sh, tmp, zacc, b2_v):
        c = lax.axis_index("c")
        s = lax.axis_index("s")
        pltpu.sync_copy(src_hbm.at[s], sidx)
        pltpu.sync_copy(dst_hbm.at[s], didx)
        pltpu.sync_copy(g2_hbm, g2_v)
        pltpu.sync_copy(b2_hbm, b2_v)
        _fill(z_v, 0.0, np_)

        @pl.loop(0, c2)
        def _(j):
            for kk in range(CH // L):
                s16 = sidx[j, pl.ds(kk * L, L)]
                d16 = didx[j, pl.ds(kk * L, L)]
                v = plsc.load_gather(g2_v, [s16])
                plsc.addupdate_scatter(z_v, [d16], v)

        base = pl.multiple_of(c * (np_ // NC) + s * rpt2, 8)
        pltpu.sync_copy(z_v, z_sh.at[s])
        plsc.subcore_barrier()
        _fill(zacc, 0.0, rpt2)

        @pl.loop(0, NS)
        def _(ss):
            pltpu.sync_copy(z_sh.at[ss, pl.ds(base, rpt2)], tmp)

            @pl.loop(0, rpt2, step=L)
            def _(k2):
                zacc[pl.ds(k2, L)] += tmp[pl.ds(k2, L)]

        pltpu.sync_copy(dis_hbm.at[pl.ds(base, rpt2)], tmp)
        b2v = b2_v[...]

        @pl.loop(0, rpt2, step=L)
        def _(k2):
            sl16 = pl.ds(k2, L)
            dis16 = tmp[sl16]
            g216 = g2_v[pl.ds(pl.multiple_of(base + k2, 8), L)]
            t16 = dis16 * zacc[sl16] + dis16 * dis16 * g216 + b2v
            zacc[sl16] = 1.0 / (1.0 + jnp.exp(-t16))

        pltpu.sync_copy(zacc, out_hbm.at[pl.ds(base, rpt2)])

    return k


def _tc_prep(dp, x, n):
    d = x.shape[1]
    nq = 2 * NC
    dq = d // nq

    def body(dp_ref, x_ref, dis_ref, gx_ref):
        deg = dp_ref[0] + dp_ref[1] + 1.0
        dis = lax.rsqrt(deg)
        dis_ref[...] = dis
        gxs = dis[:n] * x_ref[...]
        for q in range(nq):
            gx_ref[q] = gxs[:, q * dq:(q + 1) * dq]

    np_ = dp.shape[1]
    return pl.pallas_call(
        body,
        out_shape=(
            jax.ShapeDtypeStruct((np_, 1), F32),
            jax.ShapeDtypeStruct((nq, n, dq), F32),
        ),
    )(dp, x)


def _tc_mid(sp, dis, gx, w1, b1, w2, n):
    nq, np_, dq = sp.shape
    d = nq * dq
    bn = 2000
    assert n % bn == 0

    def body(sp_ref, dis_ref, gx_ref, w1_ref, b1_ref, w2_ref, g2_ref):
        s_sum = jnp.concatenate([sp_ref[q] for q in range(nq)], axis=1)
        gx_full = jnp.concatenate([gx_ref[q] for q in range(nq)], axis=1)
        dis_n = dis_ref[...]
        t = dis_n * s_sum + (dis_n * dis_n) * gx_full
        x1 = jax.nn.sigmoid(
            jnp.dot(t, w1_ref[...], preferred_element_type=F32,
                    precision=lax.Precision.HIGHEST) + b1_ref[...]
        )
        h2 = jnp.dot(x1, w2_ref[...], preferred_element_type=F32,
                     precision=lax.Precision.HIGHEST)
        g2_ref[...] = dis_n * h2

    return pl.pallas_call(
        body,
        grid=(n // bn,),
        in_specs=[
            pl.BlockSpec((nq, bn, dq), lambda i: (0, i, 0)),
            pl.BlockSpec((bn, 1), lambda i: (i, 0)),
            pl.BlockSpec((nq, bn, dq), lambda i: (0, i, 0)),
            pl.BlockSpec((d, d), lambda i: (0, 0)),
            pl.BlockSpec((1, d), lambda i: (0, 0)),
            pl.BlockSpec((d, 1), lambda i: (0, 0)),
        ],
        out_specs=pl.BlockSpec((bn, 1), lambda i: (i, 0)),
        out_shape=jax.ShapeDtypeStruct((n, 1), F32),
    )(sp, dis, gx, w1, b1, w2)


def kernel(x, edge_index, edge_attr, W1, b1, W2, b2):
    n, d = x.shape
    e = edge_index.shape[1]

    c1 = -(-e // (NC * NS * CH))
    if c1 % 2:
        c1 += 1
    e_pad = c1 * NC * NS * CH
    c2 = 2 * c1
    np_ = -(-(n + 1) // (NS * CH)) * NS * CH

    src = edge_index[0]
    dst = edge_index[1]
    if e_pad > e:
        src = jnp.concatenate([src, jnp.zeros((e_pad - e,), jnp.int32)])
        dst = jnp.concatenate([dst, jnp.full((e_pad - e,), n, jnp.int32)])
    src_r = src.reshape(NC, NS, c1, CH)
    dst_r = dst.reshape(NC, NS, c1, CH)
    src_f = src.reshape(NS, c2, CH)
    dst_f = dst.reshape(NS, c2, CH)

    deg_parts = _sc_deg(np_, c1)(dst_r)
    dis, gx = _tc_prep(deg_parts.reshape(NC, np_, 1), x, n)
    s_parts = _sc_agg(n, d, np_, c2)(gx, src_f, dst_f)
    g2 = _tc_mid(s_parts, dis, gx, W1, b1.reshape(1, d), W2, n)
    g2_pad = jnp.concatenate([g2.reshape(n), jnp.zeros((np_ - n,), F32)])
    out = _sc_agg1(n, np_, c2)(g2_pad, dis.reshape(np_),
                               jnp.broadcast_to(b2, (L,)), src_f, dst_f)
    return out[:n].reshape(n, 1)


# speedup vs baseline: 33.5331x; 1.0630x over previous
import dataclasses
import functools

import jax
import jax.numpy as jnp
from jax import lax
from jax.experimental import pallas as pl
from jax.experimental.pallas import tpu as pltpu
from jax.experimental.pallas import tpu_sc as plsc

F32 = jnp.float32

NC = 2
NS = 16
L = 16
CH = 128


def _mesh():
    return plsc.VectorSubcoreMesh(core_axis_name="c", subcore_axis_name="s")


def _no_layout_params():
    cp = pltpu.CompilerParams()
    if "needs_layout_passes" in pltpu.CompilerParams.__dataclass_fields__:
        cp = dataclasses.replace(cp, needs_layout_passes=False)
    return dataclasses.replace(cp, use_tc_tiling_on_sc=False)


def _flat_tiling_params():
    return dataclasses.replace(pltpu.CompilerParams(),
                               use_tc_tiling_on_sc=False,
                               internal_scratch_in_bytes=1024 * 1024)


def _fill(buf, value, width):
    @pl.loop(0, width, step=L)
    def _(k):
        buf[pl.ds(k, L)] = jnp.full((L,), value, F32)


def _rsqrt16(x16):
    y = plsc.bitcast(
        jnp.int32(0x5F3759DF) - (plsc.bitcast(x16, jnp.int32) >> 1), F32)
    for _ in range(3):
        y = y * (1.5 - 0.5 * x16 * y * y)
    return y


def _sc_prep(n, d, np_, c2):
    nq = 2 * NC
    dq = d // nq
    rpt = np_ // NS

    @functools.partial(
        pl.kernel,
        out_type=(
            jax.ShapeDtypeStruct((np_,), F32),
            jax.ShapeDtypeStruct((nq, np_, dq), F32),
        ),
        mesh=_mesh(),
        scratch_types=[
            pltpu.VMEM((c2, CH), jnp.int32),
            pltpu.VMEM((np_,), F32),
            pltpu.VMEM_SHARED((NS, np_), F32),
            pltpu.VMEM((rpt,), F32),
            pltpu.VMEM((rpt,), F32),
            pltpu.VMEM((rpt, dq), F32),
            pltpu.VMEM((rpt, dq), F32),
        ],
        compiler_params=_no_layout_params(),
    )
    def k(x_hbm, dst_hbm, dis_out, gx_out, didx, deg_v, z_sh, tmp, dis_v,
          xb0, xb1):
        c = lax.axis_index("c")
        s = lax.axis_index("s")
        pltpu.sync_copy(dst_hbm.at[s], didx)
        _fill(deg_v, 0.0, np_)
        ones = jnp.ones((L,), F32)

        @pl.loop(0, c2)
        def _(j):
            for kk in range(CH // L):
                d16 = didx[j, pl.ds(kk * L, L)]
                plsc.addupdate_scatter(deg_v, [d16], ones)

        base = pl.multiple_of(s * rpt, 8)
        pltpu.sync_copy(deg_v, z_sh.at[s])
        plsc.subcore_barrier()
        _fill(dis_v, 0.0, rpt)

        @pl.loop(0, NS)
        def _(ss):
            pltpu.sync_copy(z_sh.at[ss, pl.ds(base, rpt)], tmp)

            @pl.loop(0, rpt, step=L)
            def _(k2):
                dis_v[pl.ds(k2, L)] += tmp[pl.ds(k2, L)]

        @pl.loop(0, rpt, step=L)
        def _(k2):
            dis_v[pl.ds(k2, L)] = _rsqrt16(dis_v[pl.ds(k2, L)] + 1.0)

        @pl.when(c == 0)
        def _():
            pltpu.sync_copy(dis_v, dis_out.at[pl.ds(base, rpt)])

        for q, xb in ((2 * c, xb0), (2 * c + 1, xb1)):
            pltpu.sync_copy(
                x_hbm.at[pl.ds(base, rpt), pl.ds(q * dq, dq)], xb)

        @pl.loop(0, rpt)
        def _(i):
            sp16 = plsc.load_gather(dis_v, [jnp.zeros((L,), jnp.int32) + i])
            for xb in (xb0, xb1):
                for kk in range(dq // L):
                    xb[i, pl.ds(kk * L, L)] *= sp16

        for q, xb in ((2 * c, xb0), (2 * c + 1, xb1)):
            pltpu.sync_copy(xb, gx_out.at[q, pl.ds(base, rpt)])

    return k


def _sc_agg(n, d, np_, c2):
    nq = 2 * NC
    dq = d // nq
    rows_per_tile = np_ // NS
    nrt = np_ // NS

    @functools.partial(
        pl.kernel,
        out_type=jax.ShapeDtypeStruct((nq, np_, dq), F32),
        mesh=_mesh(),
        scratch_types=[
            pltpu.VMEM((c2, CH), jnp.int32),
            pltpu.VMEM((c2, CH), jnp.int32),
            pltpu.VMEM((CH, dq), F32),
            pltpu.VMEM((CH, dq), F32),
            pltpu.VMEM_SHARED((np_, dq), F32),
            pltpu.VMEM_SHARED((np_, dq), F32),
            pltpu.SemaphoreType.DMA,
            pltpu.SemaphoreType.DMA,
            pltpu.SemaphoreType.DMA,
            pltpu.SemaphoreType.DMA,
        ],
        compiler_params=_flat_tiling_params(),
    )
    def k(gx_hbm, src_hbm, dst_hbm, out_hbm, sidx, didx, rows0, rows1, gx_sp,
          acc, gsem0, gsem1, ssem0, ssem1):
        c = lax.axis_index("c")
        s = lax.axis_index("s")
        rows = (rows0, rows1)
        gsem = (gsem0, gsem1)
        ssem = (ssem0, ssem1)
        pltpu.sync_copy(src_hbm.at[s], sidx)
        pltpu.sync_copy(dst_hbm.at[s], didx)
        tsl = pl.ds(s * nrt, nrt)
        sl = pl.ds(s * rows_per_tile, rows_per_tile)

        for p in range(nq // NC):
            q = c * (nq // NC) + p
            pltpu.sync_copy(gx_hbm.at[q, tsl], gx_sp.at[tsl])

            @pl.loop(0, CH)
            def _(r):
                for kk in range(dq // L):
                    rows0[r, pl.ds(kk * L, L)] = jnp.zeros((L,), F32)

            for i in range(rows_per_tile // CH):
                pltpu.sync_copy(rows0,
                                acc.at[pl.ds(s * rows_per_tile + i * CH, CH)])
            plsc.subcore_barrier()

            for t in range(2):
                pltpu.async_copy(gx_sp.at[sidx.at[t]], rows[t], gsem[t])

            @pl.loop(0, c2, step=2)
            def _(j):
                for t in range(2):
                    jj = j + t
                    pltpu.make_async_copy(gx_sp.at[sidx.at[jj]], rows[t],
                                          gsem[t]).wait()
                    pltpu.async_copy(rows[t], acc.at[didx.at[jj]], ssem[t],
                                     add=True).wait()

                    @pl.when(jj + 2 < c2)
                    def _():
                        pltpu.async_copy(gx_sp.at[sidx.at[jj + 2]], rows[t],
                                         gsem[t])

            plsc.subcore_barrier()
            pltpu.sync_copy(acc.at[sl], out_hbm.at[q, sl])
            if p + 1 < nq // NC:
                plsc.subcore_barrier()

    return k


def _sc_agg1(n, np_, c2):
    rpt2 = np_ // (NC * NS)

    @functools.partial(
        pl.kernel,
        out_type=jax.ShapeDtypeStruct((np_,), F32),
        mesh=_mesh(),
        scratch_types=[
            pltpu.VMEM((c2, CH), jnp.int32),
            pltpu.VMEM((c2, CH), jnp.int32),
            pltpu.VMEM((np_,), F32),
            pltpu.VMEM((np_,), F32),
            pltpu.VMEM_SHARED((NS, np_), F32),
            pltpu.VMEM((rpt2,), F32),
            pltpu.VMEM((rpt2,), F32),
            pltpu.VMEM((L,), F32),
        ],
        compiler_params=_no_layout_params(),
    )
    def k(g2_hbm, dis_hbm, b2_hbm, src_hbm, dst_hbm, out_hbm, sidx, didx,
          g2_v, z_v, z_sh, tmp, zacc, b2_v):
        c = lax.axis_index("c")
        s = lax.axis_index("s")
        pltpu.sync_copy(src_hbm.at[s], sidx)
        pltpu.sync_copy(dst_hbm.at[s], didx)
        pltpu.sync_copy(g2_hbm, g2_v)
        pltpu.sync_copy(b2_hbm, b2_v)
        _fill(z_v, 0.0, np_)

        @pl.loop(0, c2)
        def _(j):
            for kk in range(CH // L):
                s16 = sidx[j, pl.ds(kk * L, L)]
                d16 = didx[j, pl.ds(kk * L, L)]
                v = plsc.load_gather(g2_v, [s16])
                plsc.addupdate_scatter(z_v, [d16], v)

        base = pl.multiple_of(c * (np_ // NC) + s * rpt2, 8)
        pltpu.sync_copy(z_v, z_sh.at[s])
        plsc.subcore_barrier()
        _fill(zacc, 0.0, rpt2)

        @pl.loop(0, NS)
        def _(ss):
            pltpu.sync_copy(z_sh.at[ss, pl.ds(base, rpt2)], tmp)

            @pl.loop(0, rpt2, step=L)
            def _(k2):
                zacc[pl.ds(k2, L)] += tmp[pl.ds(k2, L)]

        pltpu.sync_copy(dis_hbm.at[pl.ds(base, rpt2)], tmp)
        b2v = b2_v[...]

        @pl.loop(0, rpt2, step=L)
        def _(k2):
            sl16 = pl.ds(k2, L)
            dis16 = tmp[sl16]
            g216 = g2_v[pl.ds(pl.multiple_of(base + k2, 8), L)]
            t16 = dis16 * zacc[sl16] + dis16 * dis16 * g216 + b2v
            zacc[sl16] = 1.0 / (1.0 + jnp.exp(-t16))

        pltpu.sync_copy(zacc, out_hbm.at[pl.ds(base, rpt2)])

    return k


def _tc_mid(sp, dis, gx, w1, b1, w2, n):
    nq, np_, dq = sp.shape
    d = nq * dq
    bn = 2000
    assert n % bn == 0

    def body(sp_ref, dis_ref, gx_ref, w1_ref, b1_ref, w2_ref, g2_ref):
        s_sum = jnp.concatenate([sp_ref[q] for q in range(nq)], axis=1)
        gx_full = jnp.concatenate([gx_ref[q] for q in range(nq)], axis=1)
        dis_n = dis_ref[...]
        t = dis_n * s_sum + (dis_n * dis_n) * gx_full
        x1 = jax.nn.sigmoid(
            jnp.dot(t, w1_ref[...], preferred_element_type=F32,
                    precision=lax.Precision.HIGHEST) + b1_ref[...]
        )
        h2 = jnp.dot(x1, w2_ref[...], preferred_element_type=F32,
                     precision=lax.Precision.HIGHEST)
        g2_ref[...] = dis_n * h2

    return pl.pallas_call(
        body,
        grid=(n // bn,),
        in_specs=[
            pl.BlockSpec((nq, bn, dq), lambda i: (0, i, 0)),
            pl.BlockSpec((bn, 1), lambda i: (i, 0)),
            pl.BlockSpec((nq, bn, dq), lambda i: (0, i, 0)),
            pl.BlockSpec((d, d), lambda i: (0, 0)),
            pl.BlockSpec((1, d), lambda i: (0, 0)),
            pl.BlockSpec((d, 1), lambda i: (0, 0)),
        ],
        out_specs=pl.BlockSpec((bn, 1), lambda i: (i, 0)),
        out_shape=jax.ShapeDtypeStruct((n, 1), F32),
    )(sp, dis, gx, w1, b1, w2)


def kernel(x, edge_index, edge_attr, W1, b1, W2, b2):
    n, d = x.shape
    e = edge_index.shape[1]

    c1 = -(-e // (NC * NS * CH))
    if c1 % 2:
        c1 += 1
    e_pad = c1 * NC * NS * CH
    c2 = 2 * c1
    np_ = -(-(n + 1) // (NS * CH)) * NS * CH

    src = edge_index[0]
    dst = edge_index[1]
    if e_pad > e:
        src = jnp.concatenate([src, jnp.zeros((e_pad - e,), jnp.int32)])
        dst = jnp.concatenate([dst, jnp.full((e_pad - e,), n, jnp.int32)])
    src_f = src.reshape(NS, c2, CH)
    dst_f = dst.reshape(NS, c2, CH)
    x_pad = jnp.concatenate([x, jnp.zeros((np_ - n, d), F32)])

    dis, gx = _sc_prep(n, d, np_, c2)(x_pad, dst_f)
    s_parts = _sc_agg(n, d, np_, c2)(gx, src_f, dst_f)
    g2 = _tc_mid(s_parts, dis.reshape(np_, 1), gx, W1, b1.reshape(1, d),
                 W2, n)
    g2_pad = jnp.concatenate([g2.reshape(n), jnp.zeros((np_ - n,), F32)])
    out = _sc_agg1(n, np_, c2)(g2_pad, dis, jnp.broadcast_to(b2, (L,)),
                               src_f, dst_f)
    return out[:n].reshape(n, 1)


# speedup vs baseline: 34.7555x; 1.0365x over previous
import dataclasses
import functools

import jax
import jax.numpy as jnp
from jax import lax
from jax.experimental import pallas as pl
from jax.experimental.pallas import tpu as pltpu
from jax.experimental.pallas import tpu_sc as plsc

F32 = jnp.float32

NC = 2
NS = 16
L = 16
CH = 128


def _mesh():
    return plsc.VectorSubcoreMesh(core_axis_name="c", subcore_axis_name="s")


def _no_layout_params():
    cp = pltpu.CompilerParams()
    if "needs_layout_passes" in pltpu.CompilerParams.__dataclass_fields__:
        cp = dataclasses.replace(cp, needs_layout_passes=False)
    return dataclasses.replace(cp, use_tc_tiling_on_sc=False)


def _flat_tiling_params():
    return dataclasses.replace(pltpu.CompilerParams(),
                               use_tc_tiling_on_sc=False,
                               internal_scratch_in_bytes=1024 * 1024)


def _fill(buf, value, width):
    @pl.loop(0, width, step=L)
    def _(k):
        buf[pl.ds(k, L)] = jnp.full((L,), value, F32)


def _rsqrt16(x16):
    y = plsc.bitcast(
        jnp.int32(0x5F3759DF) - (plsc.bitcast(x16, jnp.int32) >> 1), F32)
    for _ in range(3):
        y = y * (1.5 - 0.5 * x16 * y * y)
    return y


def _sc_prep(n, d, np_, c2):
    nq = 2 * NC
    dq = d // nq
    rpt = np_ // NS

    @functools.partial(
        pl.kernel,
        out_type=(
            jax.ShapeDtypeStruct((np_,), F32),
            jax.ShapeDtypeStruct((nq, np_, dq), F32),
        ),
        mesh=_mesh(),
        scratch_types=[
            pltpu.VMEM((c2, CH), jnp.int32),
            pltpu.VMEM((np_,), F32),
            pltpu.VMEM_SHARED((NS, np_), F32),
            pltpu.VMEM((NS, np_ // NS), F32),
            pltpu.VMEM((np_ // NS,), F32),
            pltpu.VMEM((np_ // NS, dq), F32),
            pltpu.VMEM((np_ // NS, dq), F32),
            pltpu.SemaphoreType.DMA,
        ],
        compiler_params=_no_layout_params(),
    )
    def k(x_hbm, dst_hbm, dis_out, gx_out, didx, deg_v, z_sh, tmp, dis_v,
          xb0, xb1, rsem):
        c = lax.axis_index("c")
        s = lax.axis_index("s")
        pltpu.sync_copy(dst_hbm.at[s], didx)
        _fill(deg_v, 0.0, np_)
        ones = jnp.ones((L,), F32)

        @pl.loop(0, c2)
        def _(j):
            for kk in range(CH // L):
                d16 = didx[j, pl.ds(kk * L, L)]
                plsc.addupdate_scatter(deg_v, [d16], ones)

        base = pl.multiple_of(s * rpt, 8)
        pltpu.sync_copy(deg_v, z_sh.at[s])
        plsc.subcore_barrier()
        for ss in range(NS):
            pltpu.async_copy(z_sh.at[ss, pl.ds(base, rpt)], tmp.at[ss], rsem)
        for ss in range(NS):
            pltpu.make_async_copy(z_sh.at[ss, pl.ds(base, rpt)], tmp.at[ss],
                                  rsem).wait()

        @pl.loop(0, rpt, step=L)
        def _(k2):
            acc16 = tmp[0, pl.ds(k2, L)] + 1.0
            for ss in range(1, NS):
                acc16 += tmp[ss, pl.ds(k2, L)]
            dis_v[pl.ds(k2, L)] = _rsqrt16(acc16)

        @pl.when(c == 0)
        def _():
            pltpu.sync_copy(dis_v, dis_out.at[pl.ds(base, rpt)])

        for q, xb in ((2 * c, xb0), (2 * c + 1, xb1)):
            pltpu.sync_copy(
                x_hbm.at[pl.ds(base, rpt), pl.ds(q * dq, dq)], xb)

        @pl.loop(0, rpt)
        def _(i):
            sp16 = plsc.load_gather(dis_v, [jnp.zeros((L,), jnp.int32) + i])
            for xb in (xb0, xb1):
                for kk in range(dq // L):
                    xb[i, pl.ds(kk * L, L)] *= sp16

        for q, xb in ((2 * c, xb0), (2 * c + 1, xb1)):
            pltpu.sync_copy(xb, gx_out.at[q, pl.ds(base, rpt)])

    return k


def _sc_agg(n, d, np_, c2):
    nq = 2 * NC
    dq = d // nq
    rows_per_tile = np_ // NS
    nrt = np_ // NS

    @functools.partial(
        pl.kernel,
        out_type=jax.ShapeDtypeStruct((nq, np_, dq), F32),
        mesh=_mesh(),
        scratch_types=[
            pltpu.VMEM((c2, CH), jnp.int32),
            pltpu.VMEM((c2, CH), jnp.int32),
            pltpu.VMEM((CH, dq), F32),
            pltpu.VMEM((CH, dq), F32),
            pltpu.VMEM((CH, dq), F32),
            pltpu.VMEM((CH, dq), F32),
            pltpu.VMEM_SHARED((np_, dq), F32),
            pltpu.VMEM_SHARED((np_, dq), F32),
            pltpu.SemaphoreType.DMA,
            pltpu.SemaphoreType.DMA,
            pltpu.SemaphoreType.DMA,
            pltpu.SemaphoreType.DMA,
            pltpu.SemaphoreType.DMA,
            pltpu.SemaphoreType.DMA,
            pltpu.SemaphoreType.DMA,
            pltpu.SemaphoreType.DMA,
        ],
        compiler_params=_flat_tiling_params(),
    )
    def k(gx_hbm, src_hbm, dst_hbm, out_hbm, sidx, didx, rows0, rows1, rows2,
          rows3, gx_sp, acc, gsem0, gsem1, gsem2, gsem3, ssem0, ssem1, ssem2,
          ssem3):
        c = lax.axis_index("c")
        s = lax.axis_index("s")
        rows = (rows0, rows1, rows2, rows3)
        gsem = (gsem0, gsem1, gsem2, gsem3)
        ssem = (ssem0, ssem1, ssem2, ssem3)
        nb = 4
        pltpu.sync_copy(src_hbm.at[s], sidx)
        pltpu.sync_copy(dst_hbm.at[s], didx)
        tsl = pl.ds(s * nrt, nrt)
        sl = pl.ds(s * rows_per_tile, rows_per_tile)

        for p in range(nq // NC):
            q = c * (nq // NC) + p
            pltpu.sync_copy(gx_hbm.at[q, tsl], gx_sp.at[tsl])

            @pl.loop(0, CH)
            def _(r):
                for kk in range(dq // L):
                    rows0[r, pl.ds(kk * L, L)] = jnp.zeros((L,), F32)

            for i in range(rows_per_tile // CH):
                pltpu.sync_copy(rows0,
                                acc.at[pl.ds(s * rows_per_tile + i * CH, CH)])
            plsc.subcore_barrier()

            for t in range(nb):
                pltpu.async_copy(gx_sp.at[sidx.at[t]], rows[t], gsem[t])

            @pl.loop(0, c2, step=nb)
            def _(j):
                for t in range(nb):
                    jj = j + t
                    pltpu.make_async_copy(gx_sp.at[sidx.at[jj]], rows[t],
                                          gsem[t]).wait()
                    pltpu.async_copy(rows[t], acc.at[didx.at[jj]], ssem[t],
                                     add=True).wait()

                    @pl.when(jj + nb < c2)
                    def _():
                        pltpu.async_copy(gx_sp.at[sidx.at[jj + nb]], rows[t],
                                         gsem[t])

            plsc.subcore_barrier()
            pltpu.sync_copy(acc.at[sl], out_hbm.at[q, sl])
            if p + 1 < nq // NC:
                plsc.subcore_barrier()

    return k


def _sc_agg1(n, np_, c2):
    rpt2 = np_ // (NC * NS)

    @functools.partial(
        pl.kernel,
        out_type=jax.ShapeDtypeStruct((np_,), F32),
        mesh=_mesh(),
        scratch_types=[
            pltpu.VMEM((c2, CH), jnp.int32),
            pltpu.VMEM((c2, CH), jnp.int32),
            pltpu.VMEM((np_,), F32),
            pltpu.VMEM((np_,), F32),
            pltpu.VMEM_SHARED((NS, np_), F32),
            pltpu.VMEM((NS, rpt2), F32),
            pltpu.VMEM((rpt2,), F32),
            pltpu.VMEM((L,), F32),
            pltpu.SemaphoreType.DMA,
        ],
        compiler_params=_no_layout_params(),
    )
    def k(g2_hbm, dis_hbm, b2_hbm, src_hbm, dst_hbm, out_hbm, sidx, didx,
          g2_v, z_v, z_sh, tmp, zacc, b2_v, rsem):
        c = lax.axis_index("c")
        s = lax.axis_index("s")
        pltpu.sync_copy(src_hbm.at[s], sidx)
        pltpu.sync_copy(dst_hbm.at[s], didx)
        pltpu.sync_copy(g2_hbm, g2_v)
        pltpu.sync_copy(b2_hbm, b2_v)
        _fill(z_v, 0.0, np_)

        @pl.loop(0, c2)
        def _(j):
            for kk in range(CH // L):
                s16 = sidx[j, pl.ds(kk * L, L)]
                d16 = didx[j, pl.ds(kk * L, L)]
                v = plsc.load_gather(g2_v, [s16])
                plsc.addupdate_scatter(z_v, [d16], v)

        base = pl.multiple_of(c * (np_ // NC) + s * rpt2, 8)
        pltpu.sync_copy(z_v, z_sh.at[s])
        plsc.subcore_barrier()
        for ss in range(NS):
            pltpu.async_copy(z_sh.at[ss, pl.ds(base, rpt2)], tmp.at[ss], rsem)
        for ss in range(NS):
            pltpu.make_async_copy(z_sh.at[ss, pl.ds(base, rpt2)], tmp.at[ss],
                                  rsem).wait()
        b2v = b2_v[...]

        @pl.loop(0, rpt2, step=L)
        def _(k2):
            acc16 = tmp[0, pl.ds(k2, L)]
            for ss in range(1, NS):
                acc16 += tmp[ss, pl.ds(k2, L)]
            zacc[pl.ds(k2, L)] = acc16

        pltpu.sync_copy(dis_hbm.at[pl.ds(base, rpt2)], tmp.at[0])

        @pl.loop(0, rpt2, step=L)
        def _(k2):
            sl16 = pl.ds(k2, L)
            dis16 = tmp[0, sl16]
            g216 = g2_v[pl.ds(pl.multiple_of(base + k2, 8), L)]
            t16 = dis16 * zacc[sl16] + dis16 * dis16 * g216 + b2v
            zacc[sl16] = 1.0 / (1.0 + jnp.exp(-t16))

        pltpu.sync_copy(zacc, out_hbm.at[pl.ds(base, rpt2)])

    return k


def _tc_mid(sp, dis, gx, w1, b1, w2, n):
    nq, np_, dq = sp.shape
    d = nq * dq
    bn = 2000
    assert n % bn == 0

    def body(sp_ref, dis_ref, gx_ref, w1_ref, b1_ref, w2_ref, g2_ref):
        s_sum = jnp.concatenate([sp_ref[q] for q in range(nq)], axis=1)
        gx_full = jnp.concatenate([gx_ref[q] for q in range(nq)], axis=1)
        dis_n = dis_ref[...]
        t = dis_n * s_sum + (dis_n * dis_n) * gx_full
        x1 = jax.nn.sigmoid(
            jnp.dot(t, w1_ref[...], preferred_element_type=F32,
                    precision=lax.Precision.HIGHEST) + b1_ref[...]
        )
        h2 = jnp.dot(x1, w2_ref[...], preferred_element_type=F32,
                     precision=lax.Precision.HIGHEST)
        g2_ref[...] = dis_n * h2

    return pl.pallas_call(
        body,
        grid=(n // bn,),
        in_specs=[
            pl.BlockSpec((nq, bn, dq), lambda i: (0, i, 0)),
            pl.BlockSpec((bn, 1), lambda i: (i, 0)),
            pl.BlockSpec((nq, bn, dq), lambda i: (0, i, 0)),
            pl.BlockSpec((d, d), lambda i: (0, 0)),
            pl.BlockSpec((1, d), lambda i: (0, 0)),
            pl.BlockSpec((d, 1), lambda i: (0, 0)),
        ],
        out_specs=pl.BlockSpec((bn, 1), lambda i: (i, 0)),
        out_shape=jax.ShapeDtypeStruct((n, 1), F32),
    )(sp, dis, gx, w1, b1, w2)


def kernel(x, edge_index, edge_attr, W1, b1, W2, b2):
    n, d = x.shape
    e = edge_index.shape[1]

    c1 = -(-e // (NC * NS * CH))
    if c1 % 2:
        c1 += 1
    e_pad = c1 * NC * NS * CH
    c2 = 2 * c1
    np_ = -(-(n + 1) // (NS * CH)) * NS * CH

    src = edge_index[0]
    dst = edge_index[1]
    if e_pad > e:
        src = jnp.concatenate([src, jnp.zeros((e_pad - e,), jnp.int32)])
        dst = jnp.concatenate([dst, jnp.full((e_pad - e,), n, jnp.int32)])
    src_f = src.reshape(NS, c2, CH)
    dst_f = dst.reshape(NS, c2, CH)
    x_pad = jnp.concatenate([x, jnp.zeros((np_ - n, d), F32)])

    dis, gx = _sc_prep(n, d, np_, c2)(x_pad, dst_f)
    s_parts = _sc_agg(n, d, np_, c2)(gx, src_f, dst_f)
    g2 = _tc_mid(s_parts, dis.reshape(np_, 1), gx, W1, b1.reshape(1, d),
                 W2, n)
    g2_pad = jnp.concatenate([g2.reshape(n), jnp.zeros((np_ - n,), F32)])
    out = _sc_agg1(n, np_, c2)(g2_pad, dis, jnp.broadcast_to(b2, (L,)),
                               src_f, dst_f)
    return out[:n].reshape(n, 1)
